# baseline (speedup 1.0000x reference)
import functools

import jax
import jax.numpy as jnp
from jax import lax
from jax.experimental import pallas as pl
from jax.experimental.pallas import tpu as pltpu
from jax.experimental.pallas import tpu_sc as plsc

N_NODES = 10000
N_EDGES = 320000
D = 128

NC = 2
NS = 16
NW = NC * NS
EPW = N_EDGES // NW
K = 128
NCHUNK = N_EDGES // K
NFULL = NCHUNK // NW
NEXTRA = NCHUNK - NFULL * NW
RPT = 624
ZR = 16


def _mul_body(feat_ref, cj_ref, h_ref):
    h_ref[...] = feat_ref[...] * cj_ref[...]


def _mul_cj(feat, cj):
    blk = 2000
    grid = N_NODES // blk
    return pl.pallas_call(
        _mul_body,
        grid=(grid,),
        in_specs=[
            pl.BlockSpec((blk, D), lambda i: (i, 0)),
            pl.BlockSpec((blk, 1), lambda i: (i, 0)),
        ],
        out_specs=pl.BlockSpec((blk, D), lambda i: (i, 0)),
        out_shape=jax.ShapeDtypeStruct((N_NODES, D), jnp.float32),
    )(feat, cj)


def _sc_body(h_hbm, cj_hbm, eidx_hbm, out_hbm, ring, rows, zbuf, cj_v, acc_sh, sem_g, sem_g2, sem_i, sem_c):
    cid = lax.axis_index("c")
    sid = lax.axis_index("s")
    wid = sid * NC + cid
    nt = NFULL + jnp.where(wid < NEXTRA, 1, 0)

    def _fetch_idx(t, slot_ref):
        off = pl.multiple_of((wid + t * NW) * K, 128)
        pltpu.async_copy(eidx_hbm.at[:, pl.ds(off, K)], slot_ref, sem_i)

    KH = K // 2

    def _gather(slot, nslot_unused=None):
        del nslot_unused
        pltpu.async_copy(h_hbm.at[ring.at[slot, 0, pl.ds(0, KH)]],
                         rows.at[slot, pl.ds(0, KH)], sem_g)
        pltpu.async_copy(h_hbm.at[ring.at[slot, 0, pl.ds(KH, KH)]],
                         rows.at[slot, pl.ds(KH, KH)], sem_g2)

    def _wait_rows(slot):
        pltpu.make_async_copy(h_hbm.at[ring.at[0, 0, pl.ds(0, KH)]],
                              rows.at[slot, pl.ds(0, KH)], sem_g).wait()
        pltpu.make_async_copy(h_hbm.at[ring.at[0, 0, pl.ds(KH, KH)]],
                              rows.at[slot, pl.ds(KH, KH)], sem_g2).wait()

    def _wait_idx(slot_ref):
        pltpu.make_async_copy(eidx_hbm.at[:, pl.ds(0, K)], slot_ref, sem_i).wait()

    pltpu.async_copy(cj_hbm, cj_v, sem_c)
    pltpu.sync_copy(eidx_hbm.at[:, pl.ds(pl.multiple_of(wid * K, 128), K)],
                    ring.at[0])
    _fetch_idx(1, ring.at[1])
    _gather(0)

    def _zrow(i, _):
        for j in range(D // 16):
            zbuf[i, pl.ds(j * 16, 16)] = jnp.zeros((16,), jnp.float32)
        return ()

    lax.fori_loop(0, ZR, _zrow, ())
    row0 = sid * RPT
    ncopies = jnp.where(sid == NS - 1, (N_NODES - (NS - 1) * RPT) // ZR, RPT // ZR)

    def _zero(c, _):
        r0 = pl.multiple_of(row0 + c * ZR, 8)
        pltpu.sync_copy(zbuf, acc_sh.at[pl.ds(r0, ZR)])
        return ()

    lax.fori_loop(0, ncopies, _zero, ())
    pltpu.make_async_copy(cj_hbm, cj_v, sem_c).wait()
    plsc.subcore_barrier()

    def _chunk(t, _):
        slot = jnp.bitwise_and(t, 1)
        nslot = 1 - slot
        _wait_rows(slot)

        @pl.when(t < nt - 1)
        def _():
            _wait_idx(ring.at[nslot])
            _gather(nslot)

        def _scale(b, _):
            idxv = ring[slot, 0, pl.ds(b * 16, 16)]
            cjv = plsc.load_gather(cj_v, [idxv])
            for i in range(16):
                bc = lax.gather(
                    cjv, jnp.full((16, 1), i, jnp.int32),
                    lax.GatherDimensionNumbers(
                        offset_dims=(), collapsed_slice_dims=(0,),
                        start_index_map=(0,)),
                    (1,), mode=lax.GatherScatterMode.PROMISE_IN_BOUNDS)
                r = b * 16 + i
                for v in range(D // 16):
                    rows[slot, r, pl.ds(v * 16, 16)] = (
                        rows[slot, r, pl.ds(v * 16, 16)] * bc)
            return ()

        lax.fori_loop(0, K // 16, _scale, ())
        pltpu.sync_copy(rows.at[slot], acc_sh.at[ring.at[slot, 1]], add=True)

        @pl.when(t < nt - 2)
        def _():
            _fetch_idx(t + 2, ring.at[slot])

        return ()

    lax.fori_loop(0, nt, _chunk, ())
    plsc.subcore_barrier()

    r0 = pl.multiple_of(row0, 8)
    pltpu.sync_copy(acc_sh.at[pl.ds(r0, RPT)], out_hbm.at[cid, pl.ds(r0, RPT)])

    @pl.when(sid == NS - 1)
    def _():
        tail = (NS - 1) * RPT + RPT
        ntail = N_NODES - tail
        pltpu.sync_copy(acc_sh.at[pl.ds(tail, ntail)],
                        out_hbm.at[cid, pl.ds(tail, ntail)])


_sc_scatter = functools.partial(
    pl.kernel,
    out_type=jax.ShapeDtypeStruct((NC, N_NODES, D), jnp.float32),
    mesh=plsc.VectorSubcoreMesh(core_axis_name="c", subcore_axis_name="s"),
    compiler_params=pltpu.CompilerParams(needs_layout_passes=False),
    scratch_types=[
        pltpu.VMEM((2, 2, K), jnp.int32),
        pltpu.VMEM((2, K, D), jnp.float32),
        pltpu.VMEM((ZR, D), jnp.float32),
        pltpu.VMEM((N_NODES,), jnp.float32),
        pltpu.VMEM_SHARED((N_NODES, D), jnp.float32),
        pltpu.SemaphoreType.DMA,
        pltpu.SemaphoreType.DMA,
        pltpu.SemaphoreType.DMA,
        pltpu.SemaphoreType.DMA,
    ],
)(_sc_body)


def _comb_body(p_ref, w_ref, ci_ref, out_ref):
    s = p_ref[0] + p_ref[1]
    out_ref[...] = jnp.dot(
        s, w_ref[...], preferred_element_type=jnp.float32
    ) * ci_ref[...]


def _combine_mm(partials, weight, ci):
    blk = 1000
    grid = N_NODES // blk
    return pl.pallas_call(
        _comb_body,
        grid=(grid,),
        in_specs=[
            pl.BlockSpec((NC, blk, D), lambda i: (0, i, 0)),
            pl.BlockSpec((D, D), lambda i: (0, 0)),
            pl.BlockSpec((blk, 1), lambda i: (i, 0)),
        ],
        out_specs=pl.BlockSpec((blk, D), lambda i: (i, 0)),
        out_shape=jax.ShapeDtypeStruct((N_NODES, D), jnp.float32),
    )(partials, weight, ci)


def kernel(feat, edge_index, cj, ci, weight):
    partials = _sc_scatter(feat, cj.reshape(-1), edge_index.astype(jnp.int32))
    return _combine_mm(partials, weight, ci)


# speedup vs baseline: 3.1616x; 3.1616x over previous
import functools

import jax
import jax.numpy as jnp
from jax import lax
from jax.experimental import pallas as pl
from jax.experimental.pallas import tpu as pltpu
from jax.experimental.pallas import tpu_sc as plsc

N_NODES = 10000
N_EDGES = 320000
D = 128

NC = 2
NS = 16
NW = NC * NS
EPW = N_EDGES // NW
K = 128
NCHUNK = N_EDGES // K
NFULL = NCHUNK // NW
NEXTRA = NCHUNK - NFULL * NW
RPT = 624
ZR = 16


def _mul_body(feat_ref, cj_ref, h_ref):
    h_ref[...] = feat_ref[...] * cj_ref[...]


def _mul_cj(feat, cj):
    blk = 2000
    grid = N_NODES // blk
    return pl.pallas_call(
        _mul_body,
        grid=(grid,),
        in_specs=[
            pl.BlockSpec((blk, D), lambda i: (i, 0)),
            pl.BlockSpec((blk, 1), lambda i: (i, 0)),
        ],
        out_specs=pl.BlockSpec((blk, D), lambda i: (i, 0)),
        out_shape=jax.ShapeDtypeStruct((N_NODES, D), jnp.float32),
    )(feat, cj)


def _sc_body(h_hbm, eidx_hbm, out_hbm, ring, rows, zbuf, acc_sh, sem_g, sem_g2, sem_i):
    cid = lax.axis_index("c")
    sid = lax.axis_index("s")
    wid = sid * NC + cid
    nt = NFULL + jnp.where(wid < NEXTRA, 1, 0)

    def _fetch_idx(t, slot_ref):
        off = pl.multiple_of((wid + t * NW) * K, 128)
        pltpu.async_copy(eidx_hbm.at[:, pl.ds(off, K)], slot_ref, sem_i)

    KH = K // 2

    def _gather(slot, nslot_unused=None):
        del nslot_unused
        pltpu.async_copy(h_hbm.at[ring.at[slot, 0, pl.ds(0, KH)]],
                         rows.at[slot, pl.ds(0, KH)], sem_g)
        pltpu.async_copy(h_hbm.at[ring.at[slot, 0, pl.ds(KH, KH)]],
                         rows.at[slot, pl.ds(KH, KH)], sem_g2)

    def _wait_rows(slot):
        pltpu.make_async_copy(h_hbm.at[ring.at[0, 0, pl.ds(0, KH)]],
                              rows.at[slot, pl.ds(0, KH)], sem_g).wait()
        pltpu.make_async_copy(h_hbm.at[ring.at[0, 0, pl.ds(KH, KH)]],
                              rows.at[slot, pl.ds(KH, KH)], sem_g2).wait()

    def _wait_idx(slot_ref):
        pltpu.make_async_copy(eidx_hbm.at[:, pl.ds(0, K)], slot_ref, sem_i).wait()

    pltpu.sync_copy(eidx_hbm.at[:, pl.ds(pl.multiple_of(wid * K, 128), K)],
                    ring.at[0])
    _fetch_idx(1, ring.at[1])
    _gather(0)

    def _zrow(i, _):
        for j in range(D // 16):
            zbuf[i, pl.ds(j * 16, 16)] = jnp.zeros((16,), jnp.float32)
        return ()

    lax.fori_loop(0, ZR, _zrow, ())
    row0 = sid * RPT
    ncopies = jnp.where(sid == NS - 1, (N_NODES - (NS - 1) * RPT) // ZR, RPT // ZR)

    def _zero(c, _):
        r0 = pl.multiple_of(row0 + c * ZR, 8)
        pltpu.sync_copy(zbuf, acc_sh.at[pl.ds(r0, ZR)])
        return ()

    lax.fori_loop(0, ncopies, _zero, ())
    plsc.subcore_barrier()

    def _chunk(t, _):
        slot = jnp.bitwise_and(t, 1)
        nslot = 1 - slot
        _wait_rows(slot)

        @pl.when(t < nt - 1)
        def _():
            _wait_idx(ring.at[nslot])
            _gather(nslot)

        pltpu.sync_copy(rows.at[slot], acc_sh.at[ring.at[slot, 1]], add=True)

        @pl.when(t < nt - 2)
        def _():
            _fetch_idx(t + 2, ring.at[slot])

        return ()

    lax.fori_loop(0, nt, _chunk, ())
    plsc.subcore_barrier()

    r0 = pl.multiple_of(row0, 8)
    pltpu.sync_copy(acc_sh.at[pl.ds(r0, RPT)], out_hbm.at[cid, pl.ds(r0, RPT)])

    @pl.when(sid == NS - 1)
    def _():
        tail = (NS - 1) * RPT + RPT
        ntail = N_NODES - tail
        pltpu.sync_copy(acc_sh.at[pl.ds(tail, ntail)],
                        out_hbm.at[cid, pl.ds(tail, ntail)])


_sc_scatter = functools.partial(
    pl.kernel,
    out_type=jax.ShapeDtypeStruct((NC, N_NODES, D), jnp.float32),
    mesh=plsc.VectorSubcoreMesh(core_axis_name="c", subcore_axis_name="s"),
    scratch_types=[
        pltpu.VMEM((2, 2, K), jnp.int32),
        pltpu.VMEM((2, K, D), jnp.float32),
        pltpu.VMEM((ZR, D), jnp.float32),
        pltpu.VMEM_SHARED((N_NODES, D), jnp.float32),
        pltpu.SemaphoreType.DMA,
        pltpu.SemaphoreType.DMA,
        pltpu.SemaphoreType.DMA,
    ],
)(_sc_body)


def _comb_body(p_ref, w_ref, ci_ref, out_ref):
    s = p_ref[0] + p_ref[1]
    out_ref[...] = jnp.dot(
        s, w_ref[...], preferred_element_type=jnp.float32
    ) * ci_ref[...]


def _combine_mm(partials, weight, ci):
    blk = 1000
    grid = N_NODES // blk
    return pl.pallas_call(
        _comb_body,
        grid=(grid,),
        in_specs=[
            pl.BlockSpec((NC, blk, D), lambda i: (0, i, 0)),
            pl.BlockSpec((D, D), lambda i: (0, 0)),
            pl.BlockSpec((blk, 1), lambda i: (i, 0)),
        ],
        out_specs=pl.BlockSpec((blk, D), lambda i: (i, 0)),
        out_shape=jax.ShapeDtypeStruct((N_NODES, D), jnp.float32),
    )(partials, weight, ci)


def kernel(feat, edge_index, cj, ci, weight):
    h = _mul_cj(feat, cj)
    partials = _sc_scatter(h, edge_index.astype(jnp.int32))
    return _combine_mm(partials, weight, ci)
